# baseline (speedup 1.0000x reference)
import jax
import jax.numpy as jnp
from jax.experimental import pallas as pl
from jax.experimental.pallas import tpu as pltpu

_S = 7
_D = 30
_BS = 128
_N = _BS * _S * _S
_LAMBDA_COORD = 5.0
_LAMBDA_NOOBJ = 0.5


def _iou(px, py, pw, ph, lx, ly, lw, lh):
    p0 = px - 0.5 * pw
    p1 = py - 0.5 * ph
    p2 = px + 0.5 * pw
    p3 = py + 0.5 * ph
    l0 = lx - 0.5 * lw
    l1 = ly - 0.5 * lh
    l2 = lx + 0.5 * lw
    l3 = ly + 0.5 * lh
    mat = ~((p2 < l0) | (p0 > l2) | (p3 < l1) | (p1 > l3))
    ix0 = jnp.maximum(p0, l0)
    iy0 = jnp.maximum(p1, l1)
    ix1 = jnp.minimum(p2, l2)
    iy1 = jnp.minimum(p3, l3)
    pre_area = (p2 - p0) * (p3 - p1)
    lab_area = (l2 - l0) * (l3 - l1)
    inter = (ix1 - ix0) * (iy1 - iy0) * mat.astype(jnp.float32)
    return inter / (pre_area + lab_area - inter)


def _loss_kernel(p_ref, l_ref, out_ref):
    f32 = jnp.float32
    PT = jnp.transpose(p_ref[...].reshape(_N, _D))
    LT = jnp.transpose(l_ref[...].reshape(_N, _D))

    n = jax.lax.broadcasted_iota(jnp.int32, (1, _N), 1)
    j = (n % _S).astype(f32)
    i = ((n // _S) % _S).astype(f32)

    def box(T, b):
        return (T[5 * b + 0:5 * b + 1], T[5 * b + 1:5 * b + 2],
                T[5 * b + 2:5 * b + 3], T[5 * b + 3:5 * b + 4],
                T[5 * b + 4:5 * b + 5])

    pc0, px0, py0, pw0, ph0 = box(PT, 0)
    pc1, px1, py1, pw1, ph1 = box(PT, 1)
    lc0, lx0, ly0, lw0, lh0 = box(LT, 0)
    lc1, lx1, ly1, lw1, lh1 = box(LT, 1)

    s = f32(_S)
    iou0 = _iou((px0 + j) / s, (py0 + i) / s, pw0, ph0,
                (lx0 + j) / s, (ly0 + i) / s, lw0, lh0)
    iou1 = _iou((px1 + j) / s, (py1 + i) / s, pw1, ph1,
                (lx1 + j) / s, (ly1 + i) / s, lw1, lh1)
    take1 = iou1 > iou0

    def sel(a0, a1):
        return jnp.where(take1, a1, a0)

    mf = (lc0 == 1.0).astype(f32)

    obj_conf = jnp.sum(mf * jnp.square(sel(iou0, iou1) - sel(pc0, pc1)))

    dx = sel(lx0, lx1) - sel(px0, px1)
    dy = sel(ly0, ly1) - sel(py0, py1)
    dw = jnp.sqrt(sel(lw0, lw1)) - jnp.sqrt(sel(pw0, pw1))
    dh = jnp.sqrt(sel(lh0, lh1)) - jnp.sqrt(sel(ph0, ph1))
    obj_coord = _LAMBDA_COORD * jnp.sum(
        mf * (dx * dx + dy * dy + dw * dw + dh * dh))

    dcls = LT[10:30] - PT[10:30]
    obj_cls = jnp.sum(mf * (dcls * dcls))

    noobj1 = _LAMBDA_NOOBJ * jnp.sum(
        mf * jnp.square(sel(iou1, iou0) - sel(pc1, pc0)))
    noobj0 = _LAMBDA_NOOBJ * jnp.sum(
        (1.0 - mf) * (pc0 * pc0 + pc1 * pc1))
    noobj = noobj1 + noobj0

    obj_loss = obj_coord + obj_cls + obj_conf
    bs = f32(_BS)
    out_ref[0] = (obj_loss + noobj) / bs
    out_ref[1] = obj_cls / bs
    out_ref[2] = obj_conf / bs
    out_ref[3] = obj_coord / bs
    out_ref[4] = noobj / bs


def _run(p, l, interpret=False):
    return pl.pallas_call(
        _loss_kernel,
        out_shape=jax.ShapeDtypeStruct((5,), jnp.float32),
        out_specs=pl.BlockSpec(memory_space=pltpu.SMEM),
        interpret=interpret,
    )(p, l)


def kernel(predict, labels):
    return _run(predict, labels)


# speedup vs baseline: 4.6250x; 4.6250x over previous
import jax
import jax.numpy as jnp
from jax.experimental import pallas as pl
from jax.experimental.pallas import tpu as pltpu

_S = 7
_D = 30
_BS = 128
_G = _S * _S
_LAMBDA_COORD = 5.0
_LAMBDA_NOOBJ = 0.5


def _iou(px, py, pw, ph, lx, ly, lw, lh):
    p0 = px - 0.5 * pw
    p1 = py - 0.5 * ph
    p2 = px + 0.5 * pw
    p3 = py + 0.5 * ph
    l0 = lx - 0.5 * lw
    l1 = ly - 0.5 * lh
    l2 = lx + 0.5 * lw
    l3 = ly + 0.5 * lh
    mat = ~((p2 < l0) | (p0 > l2) | (p3 < l1) | (p1 > l3))
    ix0 = jnp.maximum(p0, l0)
    iy0 = jnp.maximum(p1, l1)
    ix1 = jnp.minimum(p2, l2)
    iy1 = jnp.minimum(p3, l3)
    pre_area = (p2 - p0) * (p3 - p1)
    lab_area = (l2 - l0) * (l3 - l1)
    inter = (ix1 - ix0) * (iy1 - iy0) * mat.astype(jnp.float32)
    return inter / (pre_area + lab_area - inter)


def _loss_kernel(p_ref, l_ref, out_ref):
    f32 = jnp.float32
    r = jax.lax.broadcasted_iota(jnp.int32, (_G, _BS), 0)
    j = (r % _S).astype(f32)
    i = (r // _S).astype(f32)

    def box(ref, b):
        return (ref[:, 5 * b + 0, :], ref[:, 5 * b + 1, :],
                ref[:, 5 * b + 2, :], ref[:, 5 * b + 3, :],
                ref[:, 5 * b + 4, :])

    pc0, px0, py0, pw0, ph0 = box(p_ref, 0)
    pc1, px1, py1, pw1, ph1 = box(p_ref, 1)
    lc0, lx0, ly0, lw0, lh0 = box(l_ref, 0)
    lc1, lx1, ly1, lw1, lh1 = box(l_ref, 1)

    s = f32(_S)
    iou0 = _iou((px0 + j) / s, (py0 + i) / s, pw0, ph0,
                (lx0 + j) / s, (ly0 + i) / s, lw0, lh0)
    iou1 = _iou((px1 + j) / s, (py1 + i) / s, pw1, ph1,
                (lx1 + j) / s, (ly1 + i) / s, lw1, lh1)
    take1 = iou1 > iou0

    def sel(a0, a1):
        return jnp.where(take1, a1, a0)

    mf = (lc0 == 1.0).astype(f32)

    obj_conf = jnp.sum(mf * jnp.square(sel(iou0, iou1) - sel(pc0, pc1)))

    dx = sel(lx0, lx1) - sel(px0, px1)
    dy = sel(ly0, ly1) - sel(py0, py1)
    dw = jnp.sqrt(sel(lw0, lw1)) - jnp.sqrt(sel(pw0, pw1))
    dh = jnp.sqrt(sel(lh0, lh1)) - jnp.sqrt(sel(ph0, ph1))
    obj_coord = _LAMBDA_COORD * jnp.sum(
        mf * (dx * dx + dy * dy + dw * dw + dh * dh))

    dcls = l_ref[:, 10:30, :] - p_ref[:, 10:30, :]
    obj_cls = jnp.sum(mf[:, None, :] * (dcls * dcls))

    noobj1 = _LAMBDA_NOOBJ * jnp.sum(
        mf * jnp.square(sel(iou1, iou0) - sel(pc1, pc0)))
    noobj0 = _LAMBDA_NOOBJ * jnp.sum(
        (1.0 - mf) * (pc0 * pc0 + pc1 * pc1))
    noobj = noobj1 + noobj0

    obj_loss = obj_coord + obj_cls + obj_conf
    bs = f32(_BS)
    out_ref[0] = (obj_loss + noobj) / bs
    out_ref[1] = obj_cls / bs
    out_ref[2] = obj_conf / bs
    out_ref[3] = obj_coord / bs
    out_ref[4] = noobj / bs


def _run(p, l, interpret=False):
    return pl.pallas_call(
        _loss_kernel,
        out_shape=jax.ShapeDtypeStruct((5,), jnp.float32),
        out_specs=pl.BlockSpec(memory_space=pltpu.SMEM),
        interpret=interpret,
    )(p, l)


def kernel(predict, labels):
    p = jnp.transpose(predict, (1, 2, 3, 0)).reshape(_G, _D, _BS)
    l = jnp.transpose(labels, (1, 2, 3, 0)).reshape(_G, _D, _BS)
    return _run(p, l)
